# Initial kernel scaffold; baseline (speedup 1.0000x reference)
import jax
import jax.numpy as jnp
from jax.experimental import pallas as pl


def kernel(z, xyz, nbr_list, boltzmannweights, atom_table, w_msg, b_msg, w_f1, b_f1, w_f2, b_f2, w_upd, b_upd, w_m1, b_m1, w_m2, b_m2, w_r1, b_r1, w_r2, b_r2):
    raise NotImplementedError("write your pallas kernel here")


# speedup vs baseline: 3.5929x; 3.5929x over previous
import functools

import jax
import jax.numpy as jnp
from jax import lax
from jax.experimental import pallas as pl
from jax.experimental.pallas import tpu as pltpu
from jax.experimental.pallas import tpu_sc as plsc

_N = 10000
_E = 320000
_D = 128
_G = 32
_NCONV = 3
_CUTOFF = 5.0
_LN2 = 0.6931471805599453

_NC = 2
_NS = 16
_NW = _NC * _NS
_EPW = _E // _NW
_CH = 80
_NCH = _EPW // _CH
_RPS = _N // _NS
_ZR = 25
_LANES = 16

_BE = 2560
_NEB = _E // _BE
_BN = 1000
_NNB = _N // _BN

_INTERP = False
_sc_params = pltpu.CompilerParams(needs_layout_passes=False,
                                  use_tc_tiling_on_sc=False)


@functools.cache
def _sc_mesh():
    return plsc.VectorSubcoreMesh(core_axis_name="c", subcore_axis_name="s",
                                  num_cores=_NC, num_subcores=_NS)


def _d2_body(xyz_hbm, a0_hbm, a1_hbm, d2_hbm, xyz_v, a0_v, a1_v, d2_v):
    c = lax.axis_index("c")
    s = lax.axis_index("s")
    w = s * _NC + c
    base = w * _EPW
    pltpu.sync_copy(xyz_hbm, xyz_v)
    pltpu.sync_copy(a0_hbm.at[pl.ds(base, _EPW)], a0_v)
    pltpu.sync_copy(a1_hbm.at[pl.ds(base, _EPW)], a1_v)

    def body(i, carry):
        sl = pl.ds(i * _LANES, _LANES)
        i0 = a0_v[sl] * 3
        i1 = a1_v[sl] * 3
        acc = jnp.zeros((_LANES,), jnp.float32)
        for comp in range(3):
            x0 = plsc.load_gather(xyz_v, [i0 + comp])
            x1 = plsc.load_gather(xyz_v, [i1 + comp])
            dx = x0 - x1
            acc = acc + dx * dx
        d2_v[sl] = acc
        return carry

    lax.fori_loop(0, _EPW // _LANES, body, 0)
    pltpu.sync_copy(d2_v, d2_hbm.at[pl.ds(base, _EPW)])


@functools.cache
def _d2_call():
    return pl.kernel(
        _d2_body,
        out_type=jax.ShapeDtypeStruct((_E,), jnp.float32),
        mesh=_sc_mesh(),
        scratch_types=[
            pltpu.VMEM((_N * 3,), jnp.float32),
            pltpu.VMEM((_EPW,), jnp.int32),
            pltpu.VMEM((_EPW,), jnp.int32),
            pltpu.VMEM((_EPW,), jnp.float32),
        ],
        compiler_params=_sc_params,
    )


def _scat_body(m_hbm, filt_hbm, a0_hbm, a1_hbm, out_hbm,
               agg_sp, a0_v, a1_v, rows_v, filt_v, stage_v, sem):
    c = lax.axis_index("c")
    s = lax.axis_index("s")
    w = s * _NC + c

    def zrow(j, carry):
        for k in range(_D // _LANES):
            stage_v[j, pl.ds(k * _LANES, _LANES)] = jnp.zeros((_LANES,), jnp.float32)
        return carry

    lax.fori_loop(0, _ZR, zrow, 0)

    def zcopy(t, carry):
        pltpu.sync_copy(stage_v, agg_sp.at[pl.ds(s * _RPS + t * _ZR, _ZR)])
        return carry

    lax.fori_loop(0, _RPS // _ZR, zcopy, 0)
    plsc.subcore_barrier()

    pltpu.sync_copy(a0_hbm.at[pl.ds(w * _NCH, _NCH)], a0_v)
    pltpu.sync_copy(a1_hbm.at[pl.ds(w * _NCH, _NCH)], a1_v)
    ebase = w * _EPW

    def chunk(g, carry):
        cp = pltpu.async_copy(m_hbm.at[a1_v.at[g]], rows_v, sem)
        pltpu.sync_copy(filt_hbm.at[pl.ds(ebase + g * _CH, _CH)], filt_v)
        cp.wait()

        def mrow(j, carry2):
            for k in range(_D // _LANES):
                sl = pl.ds(k * _LANES, _LANES)
                rows_v[j, sl] = rows_v[j, sl] * filt_v[j, sl]
            return carry2

        lax.fori_loop(0, _CH, mrow, 0)
        pltpu.sync_copy(rows_v, agg_sp.at[a0_v.at[g]], add=True)
        return carry

    lax.fori_loop(0, _NCH, chunk, 0)
    plsc.subcore_barrier()

    def drain(t, carry):
        ro = s * _RPS + t * _ZR
        pltpu.sync_copy(agg_sp.at[pl.ds(ro, _ZR)], stage_v)
        pltpu.sync_copy(stage_v, out_hbm.at[c, pl.ds(ro, _ZR)])
        return carry

    lax.fori_loop(0, _RPS // _ZR, drain, 0)


@functools.cache
def _scatter_call():
    return pl.kernel(
        _scat_body,
        out_type=jax.ShapeDtypeStruct((_NC, _N, _D), jnp.float32),
        mesh=_sc_mesh(),
        scratch_types=[
            pltpu.VMEM_SHARED((_N, _D), jnp.float32),
            pltpu.VMEM((_NCH, _CH), jnp.int32),
            pltpu.VMEM((_NCH, _CH), jnp.int32),
            pltpu.VMEM((_CH, _D), jnp.float32),
            pltpu.VMEM((_CH, _D), jnp.float32),
            pltpu.VMEM((_ZR, _D), jnp.float32),
            pltpu.SemaphoreType.DMA,
        ],
        compiler_params=_sc_params,
    )


def _filt_body(d2_ref, wf1_ref, bf1_ref, wf2_ref, bf2_ref, out_ref):
    width = _CUTOFF / (_G - 1)
    offs = lax.broadcasted_iota(jnp.int32, (1, _G), 1).astype(jnp.float32) * width
    inv_w = 1.0 / width
    d = jnp.sqrt(d2_ref[0, 0, :] + 1e-12)[:, None]
    x = (d - offs) * inv_w
    gauss = jnp.exp(-0.5 * x * x)
    h = jnp.dot(gauss, wf1_ref[0], preferred_element_type=jnp.float32)
    h = h + bf1_ref[0, 0, :][None, :]
    h = jnp.log(jnp.exp(h) + 1.0) - _LN2
    out_ref[0] = (jnp.dot(h, wf2_ref[0], preferred_element_type=jnp.float32)
                  + bf2_ref[0, 0, :][None, :])


_filt_call = pl.pallas_call(
    _filt_body,
    grid=(_NCONV, _NEB),
    in_specs=[
        pl.BlockSpec((1, 1, _BE), lambda l, e: (e, 0, 0)),
        pl.BlockSpec((1, _G, _D), lambda l, e: (l, 0, 0)),
        pl.BlockSpec((1, 1, _D), lambda l, e: (l, 0, 0)),
        pl.BlockSpec((1, _D, _D), lambda l, e: (l, 0, 0)),
        pl.BlockSpec((1, 1, _D), lambda l, e: (l, 0, 0)),
    ],
    out_specs=pl.BlockSpec((1, _BE, _D), lambda l, e: (l, e, 0)),
    out_shape=jax.ShapeDtypeStruct((_NCONV, _E, _D), jnp.float32),
    interpret=_INTERP,
)


def _embed_body(z_ref, tab_ref, wm_ref, bm_ref, r_ref, m_ref):
    z = z_ref[0, 0, :]
    oh = (z[:, None] == lax.broadcasted_iota(jnp.int32, (1, 128), 1)
          ).astype(jnp.float32)
    r = jnp.dot(oh, tab_ref[...], preferred_element_type=jnp.float32)
    r_ref[...] = r
    m_ref[...] = (jnp.dot(r, wm_ref[...], preferred_element_type=jnp.float32)
                  + bm_ref[0, :][None, :])


_embed_call = pl.pallas_call(
    _embed_body,
    grid=(_NNB,),
    in_specs=[
        pl.BlockSpec((1, 1, _BN), lambda i: (i, 0, 0)),
        pl.BlockSpec((128, _D), lambda i: (0, 0)),
        pl.BlockSpec((_D, _D), lambda i: (0, 0)),
        pl.BlockSpec((1, _D), lambda i: (0, 0)),
    ],
    out_specs=[
        pl.BlockSpec((_BN, _D), lambda i: (i, 0)),
        pl.BlockSpec((_BN, _D), lambda i: (i, 0)),
    ],
    out_shape=[
        jax.ShapeDtypeStruct((_N, _D), jnp.float32),
        jax.ShapeDtypeStruct((_N, _D), jnp.float32),
    ],
    interpret=_INTERP,
)


def _upd_core(r_ref, p_ref, wu_ref, bu_ref):
    agg = p_ref[0] + p_ref[1]
    h = jax.nn.softplus(agg) - _LN2
    return (r_ref[...] + jnp.dot(h, wu_ref[...], preferred_element_type=jnp.float32)
            + bu_ref[0, :][None, :])


def _updmsg_body(r_ref, p_ref, wu_ref, bu_ref, wm_ref, bm_ref, rout_ref, mout_ref):
    rn = _upd_core(r_ref, p_ref, wu_ref, bu_ref)
    rout_ref[...] = rn
    mout_ref[...] = (jnp.dot(rn, wm_ref[...], preferred_element_type=jnp.float32)
                     + bm_ref[0, :][None, :])


def _upd_body(r_ref, p_ref, wu_ref, bu_ref, rout_ref):
    rout_ref[...] = _upd_core(r_ref, p_ref, wu_ref, bu_ref)


_updmsg_call = pl.pallas_call(
    _updmsg_body,
    grid=(_NNB,),
    in_specs=[
        pl.BlockSpec((_BN, _D), lambda i: (i, 0)),
        pl.BlockSpec((_NC, _BN, _D), lambda i: (0, i, 0)),
        pl.BlockSpec((_D, _D), lambda i: (0, 0)),
        pl.BlockSpec((1, _D), lambda i: (0, 0)),
        pl.BlockSpec((_D, _D), lambda i: (0, 0)),
        pl.BlockSpec((1, _D), lambda i: (0, 0)),
    ],
    out_specs=[
        pl.BlockSpec((_BN, _D), lambda i: (i, 0)),
        pl.BlockSpec((_BN, _D), lambda i: (i, 0)),
    ],
    out_shape=[
        jax.ShapeDtypeStruct((_N, _D), jnp.float32),
        jax.ShapeDtypeStruct((_N, _D), jnp.float32),
    ],
    interpret=_INTERP,
)

_upd_call = pl.pallas_call(
    _upd_body,
    grid=(_NNB,),
    in_specs=[
        pl.BlockSpec((_BN, _D), lambda i: (i, 0)),
        pl.BlockSpec((_NC, _BN, _D), lambda i: (0, i, 0)),
        pl.BlockSpec((_D, _D), lambda i: (0, 0)),
        pl.BlockSpec((1, _D), lambda i: (0, 0)),
    ],
    out_specs=pl.BlockSpec((_BN, _D), lambda i: (i, 0)),
    out_shape=jax.ShapeDtypeStruct((_N, _D), jnp.float32),
    interpret=_INTERP,
)


def _readout_body(r_ref, bw_ref, wm1_ref, bm1_ref, wm2_ref, bm2_ref,
                  wr1_ref, br1_ref, wr2_ref, br2_ref, out_ref):
    ci = lax.broadcasted_iota(jnp.int32, (25 * _D, _D), 0)
    di = lax.broadcasted_iota(jnp.int32, (25 * _D, _D), 1)
    eye25 = (jnp.bitwise_and(ci, _D - 1) == di).astype(jnp.float32)
    conf = jnp.dot(r_ref[...], eye25, preferred_element_type=jnp.float32)
    h = jax.nn.softplus(
        jnp.dot(conf, wm1_ref[...], preferred_element_type=jnp.float32)
        + bm1_ref[0, :][None, :]) - _LN2
    mol = (jnp.dot(h, wm2_ref[...], preferred_element_type=jnp.float32)
           + bm2_ref[0, :][None, :])
    mi = lax.broadcasted_iota(jnp.int32, (40, 400), 0)
    ji = lax.broadcasted_iota(jnp.int32, (40, 400), 1)
    dd = ji - 10 * mi
    pool = jnp.where((dd >= 0) & (dd < 10), bw_ref[...], 0.0)
    pooled = jnp.dot(pool, mol, preferred_element_type=jnp.float32)
    h2 = jax.nn.softplus(
        jnp.dot(pooled, wr1_ref[...], preferred_element_type=jnp.float32)
        + br1_ref[0, :][None, :]) - _LN2
    logit = (jnp.dot(h2, wr2_ref[...], preferred_element_type=jnp.float32)
             + br2_ref[0, :][None, :])
    out_ref[...] = jax.nn.sigmoid(logit)


_readout_call = pl.pallas_call(
    _readout_body,
    out_shape=jax.ShapeDtypeStruct((40, 1), jnp.float32),
    interpret=_INTERP,
)


def kernel(z, xyz, nbr_list, boltzmannweights, atom_table, w_msg, b_msg,
           w_f1, b_f1, w_f2, b_f2, w_upd, b_upd, w_m1, b_m1, w_m2, b_m2,
           w_r1, b_r1, w_r2, b_r2):
    z = z.astype(jnp.int32)
    a0 = nbr_list[:, 0].astype(jnp.int32)
    a1 = nbr_list[:, 1].astype(jnp.int32)

    d2 = _d2_call()(xyz.reshape(-1), a0, a1)
    filt_all = _filt_call(d2.reshape(_NEB, 1, _BE), w_f1,
                          b_f1.reshape(_NCONV, 1, _D), w_f2,
                          b_f2.reshape(_NCONV, 1, _D))

    a0r = a0.reshape(_E // _CH, _CH)
    a1r = a1.reshape(_E // _CH, _CH)
    tab = jnp.pad(atom_table, ((0, 28), (0, 0)))
    r, m = _embed_call(z.reshape(_NNB, 1, _BN), tab, w_msg[0],
                       b_msg[0].reshape(1, _D))
    for i in range(_NCONV):
        parts = _scatter_call()(m, filt_all[i], a0r, a1r)
        if i < _NCONV - 1:
            r, m = _updmsg_call(r, parts, w_upd[i], b_upd[i].reshape(1, _D),
                                w_msg[i + 1], b_msg[i + 1].reshape(1, _D))
        else:
            r = _upd_call(r, parts, w_upd[i], b_upd[i].reshape(1, _D))

    return _readout_call(r.reshape(400, 25 * _D), boltzmannweights.reshape(1, 400), w_m1,
                         b_m1.reshape(1, -1), w_m2, b_m2.reshape(1, -1),
                         w_r1, b_r1.reshape(1, -1), w_r2, b_r2.reshape(1, 1))


# speedup vs baseline: 5.1079x; 1.4217x over previous
import functools

import jax
import jax.numpy as jnp
from jax import lax
from jax.experimental import pallas as pl
from jax.experimental.pallas import tpu as pltpu
from jax.experimental.pallas import tpu_sc as plsc

_N = 10000
_E = 320000
_D = 128
_G = 32
_NCONV = 3
_CUTOFF = 5.0
_LN2 = 0.6931471805599453

_NC = 2
_NS = 16
_NW = _NC * _NS
_EPW = _E // _NW
_CH = 80
_NCH = _EPW // _CH
_RPS = _N // _NS
_ZR = 25
_LANES = 16

_BE = 2560
_NEB = _E // _BE
_BN = 1000
_NNB = _N // _BN

_INTERP = False
_sc_params = pltpu.CompilerParams(needs_layout_passes=False,
                                  use_tc_tiling_on_sc=False)


@functools.cache
def _sc_mesh():
    return plsc.VectorSubcoreMesh(core_axis_name="c", subcore_axis_name="s",
                                  num_cores=_NC, num_subcores=_NS)


def _d2_body(xyz_hbm, a0_hbm, a1_hbm, d2_hbm, xyz_v, a0_v, a1_v, d2_v):
    c = lax.axis_index("c")
    s = lax.axis_index("s")
    w = s * _NC + c
    base = w * _EPW
    pltpu.sync_copy(xyz_hbm, xyz_v)
    pltpu.sync_copy(a0_hbm.at[pl.ds(base, _EPW)], a0_v)
    pltpu.sync_copy(a1_hbm.at[pl.ds(base, _EPW)], a1_v)

    def body(i, carry):
        sl = pl.ds(i * _LANES, _LANES)
        i0 = a0_v[sl] * 3
        i1 = a1_v[sl] * 3
        acc = jnp.zeros((_LANES,), jnp.float32)
        for comp in range(3):
            x0 = plsc.load_gather(xyz_v, [i0 + comp])
            x1 = plsc.load_gather(xyz_v, [i1 + comp])
            dx = x0 - x1
            acc = acc + dx * dx
        d2_v[sl] = acc
        return carry

    lax.fori_loop(0, _EPW // _LANES, body, 0)
    pltpu.sync_copy(d2_v, d2_hbm.at[pl.ds(base, _EPW)])


@functools.cache
def _d2_call():
    return pl.kernel(
        _d2_body,
        out_type=jax.ShapeDtypeStruct((_E,), jnp.float32),
        mesh=_sc_mesh(),
        scratch_types=[
            pltpu.VMEM((_N * 3,), jnp.float32),
            pltpu.VMEM((_EPW,), jnp.int32),
            pltpu.VMEM((_EPW,), jnp.int32),
            pltpu.VMEM((_EPW,), jnp.float32),
        ],
        compiler_params=_sc_params,
    )


def _scat_body(m_hbm, filt_hbm, a0_hbm, a1_hbm, out_hbm,
               agg_sp, a0_v, a1_v, rows_v, filt_v, stage_v, sem):
    c = lax.axis_index("c")
    s = lax.axis_index("s")
    w = s * _NC + c

    def zrow(j, carry):
        for k in range(_D // _LANES):
            stage_v[j, pl.ds(k * _LANES, _LANES)] = jnp.zeros((_LANES,), jnp.float32)
        return carry

    lax.fori_loop(0, _ZR, zrow, 0)

    def zcopy(t, carry):
        pltpu.sync_copy(stage_v, agg_sp.at[pl.ds(s * _RPS + t * _ZR, _ZR)])
        return carry

    lax.fori_loop(0, _RPS // _ZR, zcopy, 0)
    plsc.subcore_barrier()

    pltpu.sync_copy(a0_hbm.at[pl.ds(w * _NCH, _NCH)], a0_v)
    pltpu.sync_copy(a1_hbm.at[pl.ds(w * _NCH, _NCH)], a1_v)
    ebase = w * _EPW

    def chunk(g, carry):
        cp = pltpu.async_copy(m_hbm.at[a1_v.at[g]], rows_v, sem)
        pltpu.sync_copy(filt_hbm.at[pl.ds(ebase + g * _CH, _CH)], filt_v)
        cp.wait()

        def mrow(j, carry2):
            for k in range(_D // _LANES):
                sl = pl.ds(k * _LANES, _LANES)
                rows_v[j, sl] = rows_v[j, sl] * filt_v[j, sl]
            return carry2

        lax.fori_loop(0, _CH, mrow, 0)
        pltpu.sync_copy(rows_v, agg_sp.at[a0_v.at[g]], add=True)
        return carry

    lax.fori_loop(0, _NCH, chunk, 0)
    plsc.subcore_barrier()

    def drain(t, carry):
        ro = s * _RPS + t * _ZR
        pltpu.sync_copy(agg_sp.at[pl.ds(ro, _ZR)], stage_v)
        pltpu.sync_copy(stage_v, out_hbm.at[c, pl.ds(ro, _ZR)])
        return carry

    lax.fori_loop(0, _RPS // _ZR, drain, 0)


@functools.cache
def _scatter_call():
    return pl.kernel(
        _scat_body,
        out_type=jax.ShapeDtypeStruct((_NC, _N, _D), jnp.float32),
        mesh=_sc_mesh(),
        scratch_types=[
            pltpu.VMEM_SHARED((_N, _D), jnp.float32),
            pltpu.VMEM((_NCH, _CH), jnp.int32),
            pltpu.VMEM((_NCH, _CH), jnp.int32),
            pltpu.VMEM((_CH, _D), jnp.float32),
            pltpu.VMEM((_CH, _D), jnp.float32),
            pltpu.VMEM((_ZR, _D), jnp.float32),
            pltpu.SemaphoreType.DMA,
        ],
        compiler_params=_sc_params,
    )


def _filt_body(d2_ref, wf1_ref, bf1_ref, wf2_ref, bf2_ref, out_ref):
    width = _CUTOFF / (_G - 1)
    offs = lax.broadcasted_iota(jnp.int32, (1, _G), 1).astype(jnp.float32) * width
    inv_w = 1.0 / width
    d = jnp.sqrt(d2_ref[0, 0, :] + 1e-12)[:, None]
    x = (d - offs) * inv_w
    gauss = jnp.exp(-0.5 * x * x)
    h = jnp.dot(gauss, wf1_ref[...], preferred_element_type=jnp.float32)
    h = h + bf1_ref[0, :][None, :]
    h = jnp.log(jnp.exp(h) + 1.0) - _LN2
    out_ref[...] = (jnp.dot(h, wf2_ref[...], preferred_element_type=jnp.float32)
                    + bf2_ref[0, :][None, :])


_filt_call = pl.pallas_call(
    _filt_body,
    grid=(_NEB,),
    in_specs=[
        pl.BlockSpec((1, 1, _BE), lambda e: (e, 0, 0)),
        pl.BlockSpec((_G, _D), lambda e: (0, 0)),
        pl.BlockSpec((1, _D), lambda e: (0, 0)),
        pl.BlockSpec((_D, _D), lambda e: (0, 0)),
        pl.BlockSpec((1, _D), lambda e: (0, 0)),
    ],
    out_specs=pl.BlockSpec((_BE, _D), lambda e: (e, 0)),
    out_shape=jax.ShapeDtypeStruct((_E, _D), jnp.float32),
    interpret=_INTERP,
)


def _embed_body(z_ref, tab_ref, wm_ref, bm_ref, r_ref, m_ref):
    z = z_ref[0, 0, :]
    oh = (z[:, None] == lax.broadcasted_iota(jnp.int32, (1, 128), 1)
          ).astype(jnp.float32)
    r = jnp.dot(oh, tab_ref[...], preferred_element_type=jnp.float32)
    r_ref[...] = r
    m_ref[...] = (jnp.dot(r, wm_ref[...], preferred_element_type=jnp.float32)
                  + bm_ref[0, :][None, :])


_embed_call = pl.pallas_call(
    _embed_body,
    grid=(_NNB,),
    in_specs=[
        pl.BlockSpec((1, 1, _BN), lambda i: (i, 0, 0)),
        pl.BlockSpec((128, _D), lambda i: (0, 0)),
        pl.BlockSpec((_D, _D), lambda i: (0, 0)),
        pl.BlockSpec((1, _D), lambda i: (0, 0)),
    ],
    out_specs=[
        pl.BlockSpec((_BN, _D), lambda i: (i, 0)),
        pl.BlockSpec((_BN, _D), lambda i: (i, 0)),
    ],
    out_shape=[
        jax.ShapeDtypeStruct((_N, _D), jnp.float32),
        jax.ShapeDtypeStruct((_N, _D), jnp.float32),
    ],
    interpret=_INTERP,
)


def _upd_core(r_ref, p_ref, wu_ref, bu_ref):
    agg = p_ref[0] + p_ref[1]
    h = jax.nn.softplus(agg) - _LN2
    return (r_ref[...] + jnp.dot(h, wu_ref[...], preferred_element_type=jnp.float32)
            + bu_ref[0, :][None, :])


def _updmsg_body(r_ref, p_ref, wu_ref, bu_ref, wm_ref, bm_ref, rout_ref, mout_ref):
    rn = _upd_core(r_ref, p_ref, wu_ref, bu_ref)
    rout_ref[...] = rn
    mout_ref[...] = (jnp.dot(rn, wm_ref[...], preferred_element_type=jnp.float32)
                     + bm_ref[0, :][None, :])


def _upd_body(r_ref, p_ref, wu_ref, bu_ref, rout_ref):
    rout_ref[...] = _upd_core(r_ref, p_ref, wu_ref, bu_ref)


_updmsg_call = pl.pallas_call(
    _updmsg_body,
    grid=(_NNB,),
    in_specs=[
        pl.BlockSpec((_BN, _D), lambda i: (i, 0)),
        pl.BlockSpec((_NC, _BN, _D), lambda i: (0, i, 0)),
        pl.BlockSpec((_D, _D), lambda i: (0, 0)),
        pl.BlockSpec((1, _D), lambda i: (0, 0)),
        pl.BlockSpec((_D, _D), lambda i: (0, 0)),
        pl.BlockSpec((1, _D), lambda i: (0, 0)),
    ],
    out_specs=[
        pl.BlockSpec((_BN, _D), lambda i: (i, 0)),
        pl.BlockSpec((_BN, _D), lambda i: (i, 0)),
    ],
    out_shape=[
        jax.ShapeDtypeStruct((_N, _D), jnp.float32),
        jax.ShapeDtypeStruct((_N, _D), jnp.float32),
    ],
    interpret=_INTERP,
)

_upd_call = pl.pallas_call(
    _upd_body,
    grid=(_NNB,),
    in_specs=[
        pl.BlockSpec((_BN, _D), lambda i: (i, 0)),
        pl.BlockSpec((_NC, _BN, _D), lambda i: (0, i, 0)),
        pl.BlockSpec((_D, _D), lambda i: (0, 0)),
        pl.BlockSpec((1, _D), lambda i: (0, 0)),
    ],
    out_specs=pl.BlockSpec((_BN, _D), lambda i: (i, 0)),
    out_shape=jax.ShapeDtypeStruct((_N, _D), jnp.float32),
    interpret=_INTERP,
)


def _readout_body(r_ref, bw_ref, wm1_ref, bm1_ref, wm2_ref, bm2_ref,
                  wr1_ref, br1_ref, wr2_ref, br2_ref, out_ref):
    ci = lax.broadcasted_iota(jnp.int32, (25 * _D, _D), 0)
    di = lax.broadcasted_iota(jnp.int32, (25 * _D, _D), 1)
    eye25 = (jnp.bitwise_and(ci, _D - 1) == di).astype(jnp.float32)
    conf = jnp.dot(r_ref[...], eye25, preferred_element_type=jnp.float32)
    h = jax.nn.softplus(
        jnp.dot(conf, wm1_ref[...], preferred_element_type=jnp.float32)
        + bm1_ref[0, :][None, :]) - _LN2
    mol = (jnp.dot(h, wm2_ref[...], preferred_element_type=jnp.float32)
           + bm2_ref[0, :][None, :])
    mi = lax.broadcasted_iota(jnp.int32, (40, 400), 0)
    ji = lax.broadcasted_iota(jnp.int32, (40, 400), 1)
    dd = ji - 10 * mi
    pool = jnp.where((dd >= 0) & (dd < 10), bw_ref[...], 0.0)
    pooled = jnp.dot(pool, mol, preferred_element_type=jnp.float32)
    h2 = jax.nn.softplus(
        jnp.dot(pooled, wr1_ref[...], preferred_element_type=jnp.float32)
        + br1_ref[0, :][None, :]) - _LN2
    logit = (jnp.dot(h2, wr2_ref[...], preferred_element_type=jnp.float32)
             + br2_ref[0, :][None, :])
    out_ref[...] = jax.nn.sigmoid(logit)


_readout_call = pl.pallas_call(
    _readout_body,
    out_shape=jax.ShapeDtypeStruct((40, 1), jnp.float32),
    interpret=_INTERP,
)


def kernel(z, xyz, nbr_list, boltzmannweights, atom_table, w_msg, b_msg,
           w_f1, b_f1, w_f2, b_f2, w_upd, b_upd, w_m1, b_m1, w_m2, b_m2,
           w_r1, b_r1, w_r2, b_r2):
    z = z.astype(jnp.int32)
    a0 = nbr_list[:, 0].astype(jnp.int32)
    a1 = nbr_list[:, 1].astype(jnp.int32)

    d2 = _d2_call()(xyz.reshape(-1), a0, a1)
    d2r = d2.reshape(_NEB, 1, _BE)
    filts = [_filt_call(d2r, w_f1[i], b_f1[i].reshape(1, _D), w_f2[i],
                        b_f2[i].reshape(1, _D)) for i in range(_NCONV)]

    a0r = a0.reshape(_E // _CH, _CH)
    a1r = a1.reshape(_E // _CH, _CH)
    tab = jnp.pad(atom_table, ((0, 28), (0, 0)))
    r, m = _embed_call(z.reshape(_NNB, 1, _BN), tab, w_msg[0],
                       b_msg[0].reshape(1, _D))
    for i in range(_NCONV):
        parts = _scatter_call()(m, filts[i], a0r, a1r)
        if i < _NCONV - 1:
            r, m = _updmsg_call(r, parts, w_upd[i], b_upd[i].reshape(1, _D),
                                w_msg[i + 1], b_msg[i + 1].reshape(1, _D))
        else:
            r = _upd_call(r, parts, w_upd[i], b_upd[i].reshape(1, _D))

    return _readout_call(r.reshape(400, 25 * _D), boltzmannweights.reshape(1, 400), w_m1,
                         b_m1.reshape(1, -1), w_m2, b_m2.reshape(1, -1),
                         w_r1, b_r1.reshape(1, -1), w_r2, b_r2.reshape(1, 1))


# speedup vs baseline: 6.0359x; 1.1817x over previous
import functools

import jax
import jax.numpy as jnp
from jax import lax
from jax.experimental import pallas as pl
from jax.experimental.pallas import tpu as pltpu
from jax.experimental.pallas import tpu_sc as plsc

_N = 10000
_E = 320000
_D = 128
_G = 32
_NCONV = 3
_CUTOFF = 5.0
_LN2 = 0.6931471805599453

_NC = 2
_NS = 16
_NW = _NC * _NS
_EPW = _E // _NW
_CH = 40
_NCH = _EPW // _CH
_RPS = _N // _NS
_ZR = 25
_LANES = 16

_BE = 2560
_NEB = _E // _BE
_BN = 1000
_NNB = _N // _BN

_INTERP = False
_sc_params = pltpu.CompilerParams(needs_layout_passes=False,
                                  use_tc_tiling_on_sc=False)


@functools.cache
def _sc_mesh():
    return plsc.VectorSubcoreMesh(core_axis_name="c", subcore_axis_name="s",
                                  num_cores=_NC, num_subcores=_NS)


def _d2_body(xyz_hbm, a0_hbm, a1_hbm, d2_hbm, xyz_v, a0_v, a1_v, d2_v):
    c = lax.axis_index("c")
    s = lax.axis_index("s")
    w = s * _NC + c
    base = w * _EPW
    pltpu.sync_copy(xyz_hbm, xyz_v)
    pltpu.sync_copy(a0_hbm.at[pl.ds(base, _EPW)], a0_v)
    pltpu.sync_copy(a1_hbm.at[pl.ds(base, _EPW)], a1_v)

    def body(i, carry):
        sl = pl.ds(i * _LANES, _LANES)
        i0 = a0_v[sl] * 3
        i1 = a1_v[sl] * 3
        acc = jnp.zeros((_LANES,), jnp.float32)
        for comp in range(3):
            x0 = plsc.load_gather(xyz_v, [i0 + comp])
            x1 = plsc.load_gather(xyz_v, [i1 + comp])
            dx = x0 - x1
            acc = acc + dx * dx
        d2_v[sl] = acc
        return carry

    lax.fori_loop(0, _EPW // _LANES, body, 0)
    pltpu.sync_copy(d2_v, d2_hbm.at[pl.ds(base, _EPW)])


@functools.cache
def _d2_call():
    return pl.kernel(
        _d2_body,
        out_type=jax.ShapeDtypeStruct((_E,), jnp.float32),
        mesh=_sc_mesh(),
        scratch_types=[
            pltpu.VMEM((_N * 3,), jnp.float32),
            pltpu.VMEM((_EPW,), jnp.int32),
            pltpu.VMEM((_EPW,), jnp.int32),
            pltpu.VMEM((_EPW,), jnp.float32),
        ],
        compiler_params=_sc_params,
    )


def _scat_body(m_hbm, filt_hbm, a0_hbm, a1_hbm, out_hbm,
               agg_sp, a0_v, a1_v, grow0, grow1, fil0, fil1, prod0, prod1,
               sg0, sg1, sf0, sf1, ss0, ss1):
    c = lax.axis_index("c")
    s = lax.axis_index("s")
    w = s * _NC + c
    grows = (grow0, grow1)
    fils = (fil0, fil1)
    prods = (prod0, prod1)
    sgs = (sg0, sg1)
    sfs = (sf0, sf1)
    sss = (ss0, ss1)
    ebase = w * _EPW

    def zrow(j, carry):
        for k in range(_D // _LANES):
            prod0[j, pl.ds(k * _LANES, _LANES)] = jnp.zeros((_LANES,), jnp.float32)
        return carry

    lax.fori_loop(0, _ZR, zrow, 0)

    def zcopy(t, carry):
        pltpu.sync_copy(prod0.at[pl.ds(0, _ZR)],
                        agg_sp.at[pl.ds(s * _RPS + t * _ZR, _ZR)])
        return carry

    lax.fori_loop(0, _RPS // _ZR, zcopy, 0)

    pltpu.sync_copy(a0_hbm.at[pl.ds(w * _NCH, _NCH)], a0_v)
    pltpu.sync_copy(a1_hbm.at[pl.ds(w * _NCH, _NCH)], a1_v)
    plsc.subcore_barrier()

    def issue_gather(g, b):
        pltpu.async_copy(m_hbm.at[a1_v.at[g]], grows[b], sgs[b])

    def issue_filt(g, b):
        pltpu.async_copy(filt_hbm.at[pl.ds(ebase + g * _CH, _CH)],
                         fils[b], sfs[b])

    for b in range(2):
        issue_gather(b, b)
        issue_filt(b, b)

    def pair(p, carry):
        for b in range(2):
            g = 2 * p + b
            pltpu.make_async_copy(m_hbm.at[a1_v.at[g]], grows[b], sgs[b]).wait()
            pltpu.make_async_copy(filt_hbm.at[pl.ds(ebase + g * _CH, _CH)],
                                  fils[b], sfs[b]).wait()

            @pl.when(g >= 2)
            def _():
                pltpu.make_async_copy(
                    prods[b], agg_sp.at[a0_v.at[g - 2]], sss[b]).wait()

            @plsc.parallel_loop(0, _CH, 1, unroll=4)
            def mrow(j):
                for k in range(_D // 32):
                    sl = pl.ds(k * _LANES, _LANES)
                    gv = plsc.bitcast(grows[b][j, sl], jnp.bfloat16)
                    fv = plsc.bitcast(fils[b][j, sl], jnp.bfloat16)
                    glo, ghi = plsc.unpack(gv, format=plsc.PackFormat.INTERLEAVED)
                    flo, fhi = plsc.unpack(fv, format=plsc.PackFormat.INTERLEAVED)
                    prods[b][j, pl.ds(k * _LANES, _LANES)] = glo * flo
                    prods[b][j, pl.ds(64 + k * _LANES, _LANES)] = ghi * fhi

            @pl.when(g + 2 < _NCH)
            def _():
                issue_gather(g + 2, b)
                issue_filt(g + 2, b)

            pltpu.async_copy(prods[b], agg_sp.at[a0_v.at[g]], sss[b],
                             add=True)
        return carry

    lax.fori_loop(0, _NCH // 2, pair, 0)
    for b in range(2):
        g = _NCH - 2 + b
        pltpu.make_async_copy(prods[b], agg_sp.at[a0_v.at[g]], sss[b]).wait()
    plsc.subcore_barrier()

    def drain(t, carry):
        ro = s * _RPS + t * _ZR
        pltpu.sync_copy(agg_sp.at[pl.ds(ro, _ZR)], prod0.at[pl.ds(0, _ZR)])
        pltpu.sync_copy(prod0.at[pl.ds(0, _ZR)], out_hbm.at[c, pl.ds(ro, _ZR)])
        return carry

    lax.fori_loop(0, _RPS // _ZR, drain, 0)


@functools.cache
def _scatter_call():
    return pl.kernel(
        _scat_body,
        out_type=jax.ShapeDtypeStruct((_NC, _N, _D), jnp.float32),
        mesh=_sc_mesh(),
        scratch_types=[
            pltpu.VMEM_SHARED((_N, _D), jnp.float32),
            pltpu.VMEM((_NCH, _CH), jnp.int32),
            pltpu.VMEM((_NCH, _CH), jnp.int32),
            pltpu.VMEM((_CH, _D // 2), jnp.uint32),
            pltpu.VMEM((_CH, _D // 2), jnp.uint32),
            pltpu.VMEM((_CH, _D // 2), jnp.uint32),
            pltpu.VMEM((_CH, _D // 2), jnp.uint32),
            pltpu.VMEM((_CH, _D), jnp.float32),
            pltpu.VMEM((_CH, _D), jnp.float32),
            pltpu.SemaphoreType.DMA,
            pltpu.SemaphoreType.DMA,
            pltpu.SemaphoreType.DMA,
            pltpu.SemaphoreType.DMA,
            pltpu.SemaphoreType.DMA,
            pltpu.SemaphoreType.DMA,
        ],
        compiler_params=_sc_params,
    )


def _pack_bf16_pairs(f):
    lo = jax.lax.bitcast_convert_type(
        f[:, :64].astype(jnp.bfloat16), jnp.uint16).astype(jnp.uint32)
    hi = jax.lax.bitcast_convert_type(
        f[:, 64:].astype(jnp.bfloat16), jnp.uint16).astype(jnp.uint32)
    return lo | (hi << 16)


def _filt_body(d2_ref, wf1_ref, bf1_ref, wf2_ref, bf2_ref, out_ref):
    width = _CUTOFF / (_G - 1)
    offs = lax.broadcasted_iota(jnp.int32, (1, _G), 1).astype(jnp.float32) * width
    inv_w = 1.0 / width
    d = jnp.sqrt(d2_ref[0, 0, :] + 1e-12)[:, None]
    x = (d - offs) * inv_w
    gauss = jnp.exp(-0.5 * x * x)
    h = jnp.dot(gauss, wf1_ref[...], preferred_element_type=jnp.float32)
    h = h + bf1_ref[0, :][None, :]
    h = jnp.log(jnp.exp(h) + 1.0) - _LN2
    f = (jnp.dot(h, wf2_ref[...], preferred_element_type=jnp.float32)
         + bf2_ref[0, :][None, :])
    out_ref[...] = _pack_bf16_pairs(f)


_filt_call = pl.pallas_call(
    _filt_body,
    grid=(_NEB,),
    in_specs=[
        pl.BlockSpec((1, 1, _BE), lambda e: (e, 0, 0)),
        pl.BlockSpec((_G, _D), lambda e: (0, 0)),
        pl.BlockSpec((1, _D), lambda e: (0, 0)),
        pl.BlockSpec((_D, _D), lambda e: (0, 0)),
        pl.BlockSpec((1, _D), lambda e: (0, 0)),
    ],
    out_specs=pl.BlockSpec((_BE, _D // 2), lambda e: (e, 0)),
    out_shape=jax.ShapeDtypeStruct((_E, _D // 2), jnp.uint32),
    interpret=_INTERP,
)


def _embed_body(z_ref, tab_ref, wm_ref, bm_ref, r_ref, m_ref):
    z = z_ref[0, 0, :]
    oh = (z[:, None] == lax.broadcasted_iota(jnp.int32, (1, 128), 1)
          ).astype(jnp.float32)
    r = jnp.dot(oh, tab_ref[...], preferred_element_type=jnp.float32)
    r_ref[...] = r
    m_ref[...] = _pack_bf16_pairs(
        jnp.dot(r, wm_ref[...], preferred_element_type=jnp.float32)
        + bm_ref[0, :][None, :])


_embed_call = pl.pallas_call(
    _embed_body,
    grid=(_NNB,),
    in_specs=[
        pl.BlockSpec((1, 1, _BN), lambda i: (i, 0, 0)),
        pl.BlockSpec((128, _D), lambda i: (0, 0)),
        pl.BlockSpec((_D, _D), lambda i: (0, 0)),
        pl.BlockSpec((1, _D), lambda i: (0, 0)),
    ],
    out_specs=[
        pl.BlockSpec((_BN, _D), lambda i: (i, 0)),
        pl.BlockSpec((_BN, _D // 2), lambda i: (i, 0)),
    ],
    out_shape=[
        jax.ShapeDtypeStruct((_N, _D), jnp.float32),
        jax.ShapeDtypeStruct((_N, _D // 2), jnp.uint32),
    ],
    interpret=_INTERP,
)


def _upd_core(r_ref, p_ref, wu_ref, bu_ref):
    agg = p_ref[0] + p_ref[1]
    h = jax.nn.softplus(agg) - _LN2
    return (r_ref[...] + jnp.dot(h, wu_ref[...], preferred_element_type=jnp.float32)
            + bu_ref[0, :][None, :])


def _updmsg_body(r_ref, p_ref, wu_ref, bu_ref, wm_ref, bm_ref, rout_ref, mout_ref):
    rn = _upd_core(r_ref, p_ref, wu_ref, bu_ref)
    rout_ref[...] = rn
    mout_ref[...] = _pack_bf16_pairs(
        jnp.dot(rn, wm_ref[...], preferred_element_type=jnp.float32)
        + bm_ref[0, :][None, :])


def _upd_body(r_ref, p_ref, wu_ref, bu_ref, rout_ref):
    rout_ref[...] = _upd_core(r_ref, p_ref, wu_ref, bu_ref)


_updmsg_call = pl.pallas_call(
    _updmsg_body,
    grid=(_NNB,),
    in_specs=[
        pl.BlockSpec((_BN, _D), lambda i: (i, 0)),
        pl.BlockSpec((_NC, _BN, _D), lambda i: (0, i, 0)),
        pl.BlockSpec((_D, _D), lambda i: (0, 0)),
        pl.BlockSpec((1, _D), lambda i: (0, 0)),
        pl.BlockSpec((_D, _D), lambda i: (0, 0)),
        pl.BlockSpec((1, _D), lambda i: (0, 0)),
    ],
    out_specs=[
        pl.BlockSpec((_BN, _D), lambda i: (i, 0)),
        pl.BlockSpec((_BN, _D // 2), lambda i: (i, 0)),
    ],
    out_shape=[
        jax.ShapeDtypeStruct((_N, _D), jnp.float32),
        jax.ShapeDtypeStruct((_N, _D // 2), jnp.uint32),
    ],
    interpret=_INTERP,
)

_upd_call = pl.pallas_call(
    _upd_body,
    grid=(_NNB,),
    in_specs=[
        pl.BlockSpec((_BN, _D), lambda i: (i, 0)),
        pl.BlockSpec((_NC, _BN, _D), lambda i: (0, i, 0)),
        pl.BlockSpec((_D, _D), lambda i: (0, 0)),
        pl.BlockSpec((1, _D), lambda i: (0, 0)),
    ],
    out_specs=pl.BlockSpec((_BN, _D), lambda i: (i, 0)),
    out_shape=jax.ShapeDtypeStruct((_N, _D), jnp.float32),
    interpret=_INTERP,
)


def _readout_body(r_ref, bw_ref, wm1_ref, bm1_ref, wm2_ref, bm2_ref,
                  wr1_ref, br1_ref, wr2_ref, br2_ref, out_ref):
    ci = lax.broadcasted_iota(jnp.int32, (25 * _D, _D), 0)
    di = lax.broadcasted_iota(jnp.int32, (25 * _D, _D), 1)
    eye25 = (jnp.bitwise_and(ci, _D - 1) == di).astype(jnp.float32)
    conf = jnp.dot(r_ref[...], eye25, preferred_element_type=jnp.float32)
    h = jax.nn.softplus(
        jnp.dot(conf, wm1_ref[...], preferred_element_type=jnp.float32)
        + bm1_ref[0, :][None, :]) - _LN2
    mol = (jnp.dot(h, wm2_ref[...], preferred_element_type=jnp.float32)
           + bm2_ref[0, :][None, :])
    mi = lax.broadcasted_iota(jnp.int32, (40, 400), 0)
    ji = lax.broadcasted_iota(jnp.int32, (40, 400), 1)
    dd = ji - 10 * mi
    pool = jnp.where((dd >= 0) & (dd < 10), bw_ref[...], 0.0)
    pooled = jnp.dot(pool, mol, preferred_element_type=jnp.float32)
    h2 = jax.nn.softplus(
        jnp.dot(pooled, wr1_ref[...], preferred_element_type=jnp.float32)
        + br1_ref[0, :][None, :]) - _LN2
    logit = (jnp.dot(h2, wr2_ref[...], preferred_element_type=jnp.float32)
             + br2_ref[0, :][None, :])
    out_ref[...] = jax.nn.sigmoid(logit)


_readout_call = pl.pallas_call(
    _readout_body,
    out_shape=jax.ShapeDtypeStruct((40, 1), jnp.float32),
    interpret=_INTERP,
)


def kernel(z, xyz, nbr_list, boltzmannweights, atom_table, w_msg, b_msg,
           w_f1, b_f1, w_f2, b_f2, w_upd, b_upd, w_m1, b_m1, w_m2, b_m2,
           w_r1, b_r1, w_r2, b_r2):
    z = z.astype(jnp.int32)
    a0 = nbr_list[:, 0].astype(jnp.int32)
    a1 = nbr_list[:, 1].astype(jnp.int32)

    d2 = _d2_call()(xyz.reshape(-1), a0, a1)
    d2r = d2.reshape(_NEB, 1, _BE)
    filts = [_filt_call(d2r, w_f1[i], b_f1[i].reshape(1, _D), w_f2[i],
                        b_f2[i].reshape(1, _D)) for i in range(_NCONV)]

    a0r = a0.reshape(_E // _CH, _CH)
    a1r = a1.reshape(_E // _CH, _CH)
    tab = jnp.pad(atom_table, ((0, 28), (0, 0)))
    r, m = _embed_call(z.reshape(_NNB, 1, _BN), tab, w_msg[0],
                       b_msg[0].reshape(1, _D))
    for i in range(_NCONV):
        parts = _scatter_call()(m, filts[i], a0r, a1r)
        if i < _NCONV - 1:
            r, m = _updmsg_call(r, parts, w_upd[i], b_upd[i].reshape(1, _D),
                                w_msg[i + 1], b_msg[i + 1].reshape(1, _D))
        else:
            r = _upd_call(r, parts, w_upd[i], b_upd[i].reshape(1, _D))

    return _readout_call(r.reshape(400, 25 * _D), boltzmannweights.reshape(1, 400), w_m1,
                         b_m1.reshape(1, -1), w_m2, b_m2.reshape(1, -1),
                         w_r1, b_r1.reshape(1, -1), w_r2, b_r2.reshape(1, 1))
